# baseline (speedup 1.0000x reference)
import jax
import jax.numpy as jnp
from jax import lax
from jax.experimental import pallas as pl
from jax.experimental.pallas import tpu as pltpu
from jax.experimental.pallas import tpu_sc as plsc

B, L, C, F, V, D = 4096, 50, 10, 4, 100000, 32
N = B * L
K = C + F
NC, NS = 2, 16
NW = NC * NS
LN = 16
G = 128
CH = 1280
NG = CH // G
EPW = N // NW
NCH = EPW // CH


def _dst_vectors(dstb, x0, iota):
    for g in range(NG):
        e = x0 + g * G
        l = lax.shift_right_logical(e, 12)
        bb = lax.bitwise_and(e, B - 1)
        for i in range(G // LN):
            b16 = bb + (i * LN + iota)
            dstb[g, pl.ds(i * LN, LN)] = b16 * L + l


def _make_cat_body(fields):
    nf = len(fields)

    def _cat_body(cat_hbm, tab_hbm, *outs_scr):
        outs = outs_scr[:nf]
        catb, idxb, dstb, rowb, semL, semG, semS = outs_scr[nf:]
        wid = lax.axis_index("s") * NC + lax.axis_index("c")
        iota = lax.iota(jnp.int32, LN)

        def drain_scatters(prev_out):
            for q in range(NG):
                pltpu.make_async_copy(rowb.at[pl.ds(q * G, G)],
                                      prev_out.at[dstb.at[q]], semS).wait()

        def cat_chunk(ci, ch, guard):
            c = fields[ci]
            x0 = wid * EPW + ch * CH
            h = pltpu.async_copy(cat_hbm.at[pl.ds(c * N + x0, CH)], catb,
                                 semL)
            prev = outs[ci - 1] if ci > 0 else outs[nf - 1]
            if guard is None:
                drain_scatters(prev)
            else:
                @pl.when(guard)
                def _():
                    drain_scatters(prev)
            h.wait()
            for i in range(CH // LN):
                off = i * LN
                idxb[pl.ds(off, LN)] = catb[pl.ds(off, LN)] + c * V
            _dst_vectors(dstb, x0, iota)
            handles = []
            for q in range(NG):
                handles.append(pltpu.async_copy(
                    tab_hbm.at[idxb.at[pl.ds(q * G, G)]],
                    rowb.at[pl.ds(q * G, G)], semG))
            for h2 in handles:
                h2.wait()
            for q in range(NG):
                pltpu.async_copy(rowb.at[pl.ds(q * G, G)],
                                 outs[ci].at[dstb.at[q]], semS)

        def cat_loop(ch, carry):
            cat_chunk(0, ch, ch > 0)
            for ci in range(1, nf):
                cat_chunk(ci, ch, None)
            return carry

        lax.fori_loop(0, NCH, cat_loop, 0)
        drain_scatters(outs[nf - 1])

    return _cat_body


def _cont_body(cont_hbm, cw_hbm, *outs_scr):
    outs = outs_scr[:F]
    contb, cwb, dstb, rowb, semL, semS = outs_scr[F:]
    wid = lax.axis_index("s") * NC + lax.axis_index("c")
    iota = lax.iota(jnp.int32, LN)

    pltpu.sync_copy(cw_hbm, cwb)
    cwf = [[cwb[pl.ds(j * D + h * LN, LN)] for h in range(2)] for j in range(F)]

    def drain_scatters(prev_out):
        for q in range(NG):
            pltpu.make_async_copy(rowb.at[pl.ds(q * G, G)],
                                  prev_out.at[dstb.at[q]], semS).wait()

    def cont_chunk(f, ch, guard):
        x0 = wid * EPW + ch * CH
        lhandles = []
        for g in range(NG):
            e = x0 + g * G
            l = lax.shift_right_logical(e, 12)
            bb = lax.bitwise_and(e, B - 1)
            src = pl.multiple_of(l * (F * B) + f * B + bb, G)
            lhandles.append(pltpu.async_copy(
                cont_hbm.at[pl.ds(src, G)], contb.at[pl.ds(g * G, G)], semL))
        prev = outs[f - 1] if f > 0 else outs[F - 1]
        if guard is None:
            drain_scatters(prev)
        else:
            @pl.when(guard)
            def _():
                drain_scatters(prev)
        for h in lhandles:
            h.wait()
        cw_lo, cw_hi = cwf[f]

        def blk(ib, carry2):
            v16 = contb[pl.ds(ib * LN, LN)]
            for m in range(LN):
                r = ib * LN + m
                sc = v16[m]
                rowb[r, pl.ds(0, LN)] = cw_lo * sc
                rowb[r, pl.ds(LN, LN)] = cw_hi * sc
            return carry2

        lax.fori_loop(0, CH // LN, blk, 0)
        _dst_vectors(dstb, x0, iota)
        for q in range(NG):
            pltpu.async_copy(rowb.at[pl.ds(q * G, G)],
                             outs[f].at[dstb.at[q]], semS)

    def cont_loop(ch, carry):
        cont_chunk(0, ch, ch > 0)
        for f in range(1, F):
            cont_chunk(f, ch, None)
        return carry

    lax.fori_loop(0, NCH, cont_loop, 0)
    drain_scatters(outs[F - 1])


@jax.jit
def _run(catf, contf, tabs, cont_W):
    mesh = plsc.VectorSubcoreMesh(core_axis_name="c", subcore_axis_name="s")
    cont_outs = pl.kernel(
        _cont_body,
        out_type=tuple(jax.ShapeDtypeStruct((N, D), jnp.float32)
                       for _ in range(F)),
        mesh=mesh,
        compiler_params=pltpu.CompilerParams(use_tc_tiling_on_sc=False),
        scratch_types=[
            pltpu.VMEM((CH,), jnp.float32),
            pltpu.VMEM((F * D,), jnp.float32),
            pltpu.VMEM((NG, G), jnp.int32),
            pltpu.VMEM((CH, D), jnp.float32),
            pltpu.SemaphoreType.DMA,
            pltpu.SemaphoreType.DMA,
        ],
    )(contf, cont_W)
    cat_outs = []
    for fields in ((0, 1), (2, 3, 4, 5), (6, 7, 8, 9)):
        cat_outs.extend(pl.kernel(
            _make_cat_body(fields),
            out_type=tuple(jax.ShapeDtypeStruct((N, D), jnp.float32)
                           for _ in fields),
            mesh=mesh,
            compiler_params=pltpu.CompilerParams(use_tc_tiling_on_sc=False),
            scratch_types=[
                pltpu.VMEM((CH,), jnp.int32),
                pltpu.VMEM((CH,), jnp.int32),
                pltpu.VMEM((NG, G), jnp.int32),
                pltpu.VMEM((CH, D), jnp.float32),
                pltpu.SemaphoreType.DMA,
                pltpu.SemaphoreType.DMA,
                pltpu.SemaphoreType.DMA,
            ],
        )(catf, tabs))
    return tuple(cat_outs) + cont_outs


def kernel(cat, cont, emb_tables, cont_W):
    catf = jnp.transpose(cat, (2, 1, 0)).reshape(C * L * B).astype(jnp.int32)
    contf = jnp.transpose(cont, (1, 2, 0)).reshape(L * F * B)
    tabs = emb_tables.reshape(C * V, D)
    cwf = cont_W.reshape(F * D)
    outs = _run(catf, contf, tabs, cwf)
    return jnp.transpose(jnp.stack(outs, axis=0), (1, 0, 2))
